# baseline (speedup 1.0000x reference)
import functools
import math

import jax
import jax.numpy as jnp
from jax import lax
from jax.experimental import pallas as pl
from jax.experimental.pallas import tpu as pltpu
from jax.experimental.pallas import tpu_sc as plsc

_D = 768
_S = 4096
_B = 4
_NW = 32
_PW = _S // _NW
_CP = 8
_NCH = _PW // _CP
_ROWS = _B * _CP
_NJ = _D // 16
_NSLOT = 2
_NGRP = _NCH // _NSLOT
_SCALE = math.sqrt(float(_D))


def _pos_encoding():
    position = jnp.arange(_S, dtype=jnp.float32)
    num_timescales = _D // 2
    log_timescale = math.log(10000.0) / (float(num_timescales) - 1.0)
    div_terms = jnp.exp(
        jnp.arange(num_timescales, dtype=jnp.float32) * -log_timescale)
    scaled_time = position[:, None] * div_terms[None, :]
    return jnp.concatenate(
        [jnp.sin(scaled_time), jnp.cos(scaled_time)], axis=1)


def _sc_body(idx_hbm, table_hbm, pe_hbm, out_hbm, idx_v,
             e0, e1, u0, u1, p0, p1,
             g0, g1, q0, q1, o0, o1):
    emb = [e0, e1]
    outb = [u0, u1]
    peb = [p0, p1]
    gsem = [g0, g1]
    psem = [q0, q1]
    osem = [o0, o1]
    wid = lax.axis_index("s") * 2 + lax.axis_index("c")
    pltpu.sync_copy(idx_hbm.at[pl.ds(wid * (_B * _PW), _B * _PW)], idx_v)

    def start_in(c, s):
        pltpu.async_copy(
            table_hbm.at[idx_v.at[pl.ds(c * _ROWS, _ROWS)]], emb[s], gsem[s])
        pltpu.async_copy(
            pe_hbm.at[pl.ds(wid * _PW + c * _CP, _CP)], peb[s], psem[s])

    def wait_in(s):
        pltpu.make_async_copy(
            table_hbm.at[idx_v.at[pl.ds(0, _ROWS)]], emb[s], gsem[s]).wait()
        pltpu.make_async_copy(
            pe_hbm.at[pl.ds(0, _CP)], peb[s], psem[s]).wait()

    def start_out(c, s):
        for b in range(_B):
            pltpu.async_copy(
                outb[s].at[pl.ds(b * _CP, _CP)],
                out_hbm.at[pl.ds(b * _S + wid * _PW + c * _CP, _CP)],
                osem[s])

    def wait_out(s):
        for b in range(_B):
            pltpu.make_async_copy(
                outb[s].at[pl.ds(b * _CP, _CP)],
                out_hbm.at[pl.ds(0, _CP)], osem[s]).wait()

    for s in range(_NSLOT):
        start_in(s, s)

    def group_body(g, carry):
        for s in range(_NSLOT):
            c = g * _NSLOT + s
            wait_in(s)

            @pl.when(g > 0)
            def _drain(s=s):
                wait_out(s)

            def pos_body(p, carry2, s=s):
                for j in range(_NJ):
                    col = j * 16
                    pv = peb[s][p, pl.ds(col, 16)]
                    for b in range(_B):
                        r = b * _CP + p
                        outb[s][r, pl.ds(col, 16)] = (
                            emb[s][r, pl.ds(col, 16)] * _SCALE + pv)
                return carry2

            lax.fori_loop(0, _CP, pos_body, 0)
            start_out(c, s)

            @pl.when(g < _NGRP - 1)
            def _refill(g=g, s=s):
                start_in((g + 1) * _NSLOT + s, s)
        return carry

    lax.fori_loop(0, _NGRP, group_body, 0)
    for s in range(_NSLOT):
        wait_out(s)


_sc_call = pl.kernel(
    _sc_body,
    out_type=jax.ShapeDtypeStruct((_B * _S, _D), jnp.float32),
    mesh=plsc.VectorSubcoreMesh(core_axis_name="c", subcore_axis_name="s"),
    scratch_types=(
        [pltpu.VMEM((_B * _PW,), jnp.int32)]
        + [pltpu.VMEM((_ROWS, _D), jnp.float32)] * (2 * _NSLOT)
        + [pltpu.VMEM((_CP, _D), jnp.float32)] * _NSLOT
        + [pltpu.SemaphoreType.DMA] * (3 * _NSLOT)
    ),
)


def kernel(x, table, training):
    del training
    idx = (x.astype(jnp.int32)
           .reshape(_B, _NW, _NCH, _CP)
           .transpose(1, 2, 0, 3)
           .reshape(-1))
    pe = _pos_encoding()
    out = _sc_call(idx, table, pe)
    return out.reshape(_B, _S, _D)


# speedup vs baseline: 1.9581x; 1.9581x over previous
import functools
import math

import jax
import jax.numpy as jnp
import numpy as np
from jax import lax
from jax.experimental import pallas as pl
from jax.experimental.pallas import tpu as pltpu
from jax.experimental.pallas import tpu_sc as plsc

_D = 768
_S = 4096
_B = 4
_NW = 32
_PW = _S // _NW
_CP = 8
_NCH = _PW // _CP
_ROWS = _B * _CP
_NJ = _D // 16
_NSLOT = 4
_NGRP = _NCH // _NSLOT
_SCALE = math.sqrt(float(_D))


def _pos_encoding_np():
    position = np.arange(_S, dtype=np.float32)
    num_timescales = _D // 2
    log_timescale = math.log(10000.0) / (float(num_timescales) - 1.0)
    div_terms = np.exp(
        np.arange(num_timescales, dtype=np.float32) * -log_timescale)
    scaled_time = position[:, None] * div_terms[None, :]
    return np.concatenate(
        [np.sin(scaled_time), np.cos(scaled_time)], axis=1)


_PE_CONST = _pos_encoding_np()


def _sc_body(x_hbm, table_hbm, pe_hbm, out_hbm, idx_v,
             e0, e1, e2, e3, p0, p1, p2, p3,
             g0, g1, g2, g3, q0, q1, q2, q3, o0, o1, o2, o3):
    emb = [e0, e1, e2, e3]
    peb = [p0, p1, p2, p3]
    gsem = [g0, g1, g2, g3]
    psem = [q0, q1, q2, q3]
    osem = [o0, o1, o2, o3]
    wid = lax.axis_index("s") * 2 + lax.axis_index("c")
    for b in range(_B):
        pltpu.sync_copy(x_hbm.at[pl.ds(b * _S + wid * _PW, _PW)],
                        idx_v.at[pl.ds(b * _PW, _PW)])

    def start_io(c, s):
        for b in range(_B):
            pltpu.async_copy(
                table_hbm.at[idx_v.at[pl.ds(b * _PW + c * _CP, _CP)]],
                emb[s].at[pl.ds(b * _CP, _CP)], gsem[s])
        pltpu.async_copy(
            pe_hbm.at[pl.ds(wid * _PW + c * _CP, _CP)], peb[s], psem[s])

    def wait_in(s):
        for b in range(_B):
            pltpu.make_async_copy(
                table_hbm.at[idx_v.at[pl.ds(0, _CP)]],
                emb[s].at[pl.ds(b * _CP, _CP)], gsem[s]).wait()
        pltpu.make_async_copy(
            pe_hbm.at[pl.ds(0, _CP)], peb[s], psem[s]).wait()

    for s in range(_NSLOT):
        start_io(s, s)

    def group_body(g, carry):
        for s in range(_NSLOT):
            c = g * _NSLOT + s
            wait_in(s)

            def pos_body(p, carry2, s=s):
                for j in range(_NJ):
                    col = j * 16
                    pv = peb[s][p, pl.ds(col, 16)]
                    for b in range(_B):
                        r = b * _CP + p
                        emb[s][r, pl.ds(col, 16)] = (
                            emb[s][r, pl.ds(col, 16)] * _SCALE + pv)
                return carry2

            lax.fori_loop(0, _CP, pos_body, 0)
            for b in range(_B):
                pltpu.async_copy(
                    emb[s].at[pl.ds(b * _CP, _CP)],
                    out_hbm.at[pl.ds(b * _S + wid * _PW + c * _CP, _CP)],
                    osem[s])
        for s in range(_NSLOT):
            for b in range(_B):
                pltpu.make_async_copy(
                    emb[s].at[pl.ds(b * _CP, _CP)],
                    out_hbm.at[pl.ds(0, _CP)], osem[s]).wait()

            @pl.when(g < _NGRP - 1)
            def _refill(g=g, s=s):
                start_io((g + 1) * _NSLOT + s, s)
        return carry

    lax.fori_loop(0, _NGRP, group_body, 0)


_sc_call = pl.kernel(
    _sc_body,
    out_type=jax.ShapeDtypeStruct((_B * _S, _D), jnp.float32),
    mesh=plsc.VectorSubcoreMesh(core_axis_name="c", subcore_axis_name="s"),
    scratch_types=(
        [pltpu.VMEM((_B * _PW,), jnp.int32)]
        + [pltpu.VMEM((_ROWS, _D), jnp.float32)] * _NSLOT
        + [pltpu.VMEM((_CP, _D), jnp.float32)] * _NSLOT
        + [pltpu.SemaphoreType.DMA] * (3 * _NSLOT)
    ),
)


def kernel(x, table, training):
    del training
    x_flat = x.astype(jnp.int32).reshape(-1)
    pe = jnp.asarray(_PE_CONST)
    out = _sc_call(x_flat, table, pe)
    return out.reshape(_B, _S, _D)


# speedup vs baseline: 2.0072x; 1.0251x over previous
import functools
import math

import jax
import jax.numpy as jnp
import numpy as np
from jax import lax
from jax.experimental import pallas as pl
from jax.experimental.pallas import tpu as pltpu
from jax.experimental.pallas import tpu_sc as plsc

_D = 768
_S = 4096
_B = 4
_NW = 32
_PW = _S // _NW
_CP = 8
_NCH = _PW // _CP
_ROWS = _B * _CP
_NJ = _D // 16
_NSLOT = 4
_NGRP = _NCH // _NSLOT
_SCALE = math.sqrt(float(_D))


def _pos_encoding_np():
    position = np.arange(_S, dtype=np.float32)
    num_timescales = _D // 2
    log_timescale = math.log(10000.0) / (float(num_timescales) - 1.0)
    div_terms = np.exp(
        np.arange(num_timescales, dtype=np.float32) * -log_timescale)
    scaled_time = position[:, None] * div_terms[None, :]
    return np.concatenate(
        [np.sin(scaled_time), np.cos(scaled_time)], axis=1)


_PE_CONST = _pos_encoding_np()


def _sc_body(x_hbm, table_hbm, pe_hbm, out_hbm, idx_v,
             e0, e1, e2, e3, p0, p1, p2, p3,
             g0, g1, g2, g3, q0, q1, q2, q3, o0, o1, o2, o3):
    emb = [e0, e1, e2, e3]
    peb = [p0, p1, p2, p3]
    gsem = [g0, g1, g2, g3]
    psem = [q0, q1, q2, q3]
    osem = [o0, o1, o2, o3]
    wid = lax.axis_index("s") * 2 + lax.axis_index("c")
    for b in range(_B):
        pltpu.sync_copy(x_hbm.at[b, pl.ds(wid * _PW, _PW)],
                        idx_v.at[pl.ds(b * _PW, _PW)])

    def start_io(c, s):
        for b in range(_B):
            pltpu.async_copy(
                table_hbm.at[idx_v.at[pl.ds(b * _PW + c * _CP, _CP)]],
                emb[s].at[pl.ds(b * _CP, _CP)], gsem[s])
        pltpu.async_copy(
            pe_hbm.at[pl.ds(wid * _PW + c * _CP, _CP)], peb[s], psem[s])

    def wait_in(s):
        for b in range(_B):
            pltpu.make_async_copy(
                table_hbm.at[idx_v.at[pl.ds(0, _CP)]],
                emb[s].at[pl.ds(b * _CP, _CP)], gsem[s]).wait()
        pltpu.make_async_copy(
            pe_hbm.at[pl.ds(0, _CP)], peb[s], psem[s]).wait()

    for s in range(_NSLOT):
        start_io(s, s)

    def group_body(g, carry):
        for s in range(_NSLOT):
            c = g * _NSLOT + s
            wait_in(s)

            def pos_body(p, carry2, s=s):
                for j in range(_NJ):
                    col = j * 16
                    pv = peb[s][p, pl.ds(col, 16)]
                    for b in range(_B):
                        r = b * _CP + p
                        emb[s][r, pl.ds(col, 16)] = (
                            emb[s][r, pl.ds(col, 16)] * _SCALE + pv)
                return carry2

            lax.fori_loop(0, _CP, pos_body, 0)
            for b in range(_B):
                pltpu.async_copy(
                    emb[s].at[pl.ds(b * _CP, _CP)],
                    out_hbm.at[b, pl.ds(wid * _PW + c * _CP, _CP)],
                    osem[s])
        for s in range(_NSLOT):
            for b in range(_B):
                pltpu.make_async_copy(
                    emb[s].at[pl.ds(b * _CP, _CP)],
                    out_hbm.at[b, pl.ds(0, _CP)], osem[s]).wait()

            @pl.when(g < _NGRP - 1)
            def _refill(g=g, s=s):
                start_io((g + 1) * _NSLOT + s, s)
        return carry

    lax.fori_loop(0, _NGRP, group_body, 0)


_sc_call = pl.kernel(
    _sc_body,
    out_type=jax.ShapeDtypeStruct((_B, _S, _D), jnp.float32),
    mesh=plsc.VectorSubcoreMesh(core_axis_name="c", subcore_axis_name="s"),
    scratch_types=(
        [pltpu.VMEM((_B * _PW,), jnp.int32)]
        + [pltpu.VMEM((_ROWS, _D), jnp.float32)] * _NSLOT
        + [pltpu.VMEM((_CP, _D), jnp.float32)] * _NSLOT
        + [pltpu.SemaphoreType.DMA] * (3 * _NSLOT)
    ),
)


def kernel(x, table, training):
    del training
    pe = jnp.asarray(_PE_CONST)
    return _sc_call(x.astype(jnp.int32), table, pe)
